# baseline (speedup 1.0000x reference)
import functools

import jax
import jax.numpy as jnp
from jax import lax
from jax.experimental import pallas as pl
from jax.experimental.pallas import tpu as pltpu, tpu_sc as plsc

D_MODEL = 64
GATHER = 128


@functools.lru_cache(maxsize=None)
def _build(B: int, D: int):
    info = plsc.get_sparse_core_info()
    NC, NS = info.num_cores, info.num_subcores
    NW = NC * NS
    assert B % (8 * NW) == 0
    b_per_w = B // NW
    assert b_per_w % GATHER == 0
    n_steps = b_per_w // GATHER

    mesh = plsc.VectorSubcoreMesh(core_axis_name="c", subcore_axis_name="s")

    @functools.partial(
        pl.kernel,
        out_type=jax.ShapeDtypeStruct((B, D), jnp.float32),
        mesh=mesh,
        scratch_types=[
            pltpu.VMEM((b_per_w,), jnp.int32),
            pltpu.VMEM((GATHER, D), jnp.float32),
            pltpu.SemaphoreType.DMA,
        ],
        compiler_params=pltpu.CompilerParams(use_tc_tiling_on_sc=False),
    )
    def emb(table_hbm, idx_hbm, out_hbm, idx_v, rows_v, sem):
        wid = lax.axis_index("s") * NC + lax.axis_index("c")
        base = wid * b_per_w
        pltpu.sync_copy(idx_hbm.at[pl.ds(base, b_per_w)], idx_v)

        def step(i, carry):
            off = i * GATHER
            pltpu.async_copy(
                table_hbm.at[idx_v.at[pl.ds(off, GATHER)]], rows_v, sem
            ).wait()
            pltpu.sync_copy(rows_v, out_hbm.at[pl.ds(base + off, GATHER)])
            return carry

        lax.fori_loop(0, n_steps, step, 0)

    return emb


def kernel(x, table):
    orig_shape = x.shape
    xf = x.reshape(-1).astype(jnp.int32)
    out = _build(xf.shape[0], table.shape[1])(table, xf)
    return out.reshape(*orig_shape, table.shape[1])


# speedup vs baseline: 1.1177x; 1.1177x over previous
import functools

import jax
import jax.numpy as jnp
from jax import lax
from jax.experimental import pallas as pl
from jax.experimental.pallas import tpu as pltpu, tpu_sc as plsc

GATHER = 128
NBUF = 4


@functools.lru_cache(maxsize=None)
def _build(B: int, D: int):
    info = plsc.get_sparse_core_info()
    NC, NS = info.num_cores, info.num_subcores
    NW = NC * NS
    assert B % (NW * GATHER * NBUF) == 0
    b_per_w = B // NW
    n_steps = b_per_w // GATHER
    n_groups = n_steps // NBUF

    mesh = plsc.VectorSubcoreMesh(core_axis_name="c", subcore_axis_name="s")

    @functools.partial(
        pl.kernel,
        out_type=jax.ShapeDtypeStruct((B, D), jnp.float32),
        mesh=mesh,
        scratch_types=[
            pltpu.VMEM((b_per_w,), jnp.int32),
            pltpu.VMEM((NBUF, GATHER, D), jnp.float32),
        ] + [pltpu.SemaphoreType.DMA] * NBUF,
        compiler_params=pltpu.CompilerParams(use_tc_tiling_on_sc=False),
    )
    def emb(table_hbm, idx_hbm, out_hbm, idx_v, rows_v, *sems):
        wid = lax.axis_index("s") * NC + lax.axis_index("c")
        base = wid * b_per_w
        pltpu.sync_copy(idx_hbm.at[pl.ds(base, b_per_w)], idx_v)

        def fire(step, b):
            pltpu.async_copy(
                table_hbm.at[idx_v.at[pl.ds(step * GATHER, GATHER)]],
                rows_v.at[b],
                sems[b],
            )

        def wait_and_write(step, b):
            pltpu.make_async_copy(
                table_hbm.at[idx_v.at[pl.ds(step * GATHER, GATHER)]],
                rows_v.at[b],
                sems[b],
            ).wait()
            pltpu.sync_copy(
                rows_v.at[b], out_hbm.at[pl.ds(base + step * GATHER, GATHER)]
            )

        for b in range(NBUF):
            fire(b, b)

        @pl.loop(0, n_groups - 1)
        def grp(k):
            for b in range(NBUF):
                i = k * NBUF + b
                wait_and_write(i, b)
                fire(i + NBUF, b)

        for b in range(NBUF):
            wait_and_write((n_groups - 1) * NBUF + b, b)

    return emb


def kernel(x, table):
    orig_shape = x.shape
    xf = x.reshape(-1).astype(jnp.int32)
    out = _build(xf.shape[0], table.shape[1])(table, xf)
    return out.reshape(*orig_shape, table.shape[1])
